# baseline (speedup 1.0000x reference)
import functools
import jax
import jax.numpy as jnp
from jax import lax
from jax.experimental import pallas as pl
from jax.experimental.pallas import tpu as pltpu
from jax.experimental.pallas import tpu_sc as plsc

S_BLK = 512
NC = 2
NS = 16
NW = NC * NS
L = 16
CH = 16


def _add_pe_kernel(x_ref, pe_ref, o_ref):
    o_ref[...] = x_ref[...] + pe_ref[...][None, :, :]


def _tc_part(x, pe_weight, nb):
    B, S, D = x.shape
    grid = (S // S_BLK,)
    return pl.pallas_call(
        _add_pe_kernel,
        grid=grid,
        in_specs=[
            pl.BlockSpec((nb, S_BLK, D), lambda i: (0, i, 0)),
            pl.BlockSpec((S_BLK, D), lambda i: (i, 0)),
        ],
        out_specs=pl.BlockSpec((nb, S_BLK, D), lambda i: (0, i, 0)),
        out_shape=jax.ShapeDtypeStruct((nb, S, D), x.dtype),
    )(x, pe_weight)


def _sc_part(x2, pe_weight, row_base, S, D):
    seq_per_w = S // NW
    njobs = seq_per_w // CH
    mesh = plsc.VectorSubcoreMesh(core_axis_name="c", subcore_axis_name="s")

    @functools.partial(
        pl.kernel,
        out_type=jax.ShapeDtypeStruct((S, D), jnp.float32),
        mesh=mesh,
        scratch_types=[
            pltpu.VMEM((3, CH, D), jnp.float32),
            pltpu.VMEM((3, CH, D), jnp.float32),
            pltpu.SemaphoreType.DMA((3,)),
            pltpu.SemaphoreType.DMA((3,)),
            pltpu.SemaphoreType.DMA((3,)),
        ],
    )
    def k(x_hbm, pe_hbm, out_hbm, xbuf, pebuf, load_sem, pe_sem, store_sem):
        wid = lax.axis_index("s") * NC + lax.axis_index("c")
        s0 = wid * seq_per_w

        for i in range(njobs + 1):
            if i < njobs:
                sl = i % 3
                if i >= 3:
                    pltpu.make_async_copy(
                        xbuf.at[sl], out_hbm.at[pl.ds(s0 + (i - 3) * CH, CH)],
                        store_sem.at[sl]).wait()
                pltpu.async_copy(
                    x_hbm.at[pl.ds(row_base + s0 + i * CH, CH)], xbuf.at[sl],
                    load_sem.at[sl])
                pltpu.async_copy(
                    pe_hbm.at[pl.ds(s0 + i * CH, CH)], pebuf.at[sl],
                    pe_sem.at[sl])
            if i >= 1:
                j = i - 1
                sl = j % 3
                pltpu.make_async_copy(
                    x_hbm.at[pl.ds(row_base + s0 + j * CH, CH)], xbuf.at[sl],
                    load_sem.at[sl]).wait()
                pltpu.make_async_copy(
                    pe_hbm.at[pl.ds(s0 + j * CH, CH)], pebuf.at[sl],
                    pe_sem.at[sl]).wait()

                def body(r, carry):
                    def inner(q, carry2):
                        for u in range(32):
                            d0 = (q * 32 + u) * L
                            xbuf[sl, r, pl.ds(d0, L)] = (
                                xbuf[sl, r, pl.ds(d0, L)]
                                + pebuf[sl, r, pl.ds(d0, L)])
                        return carry2
                    return lax.fori_loop(0, D // L // 32, inner, carry)

                lax.fori_loop(0, CH, body, 0)
                pltpu.async_copy(
                    xbuf.at[sl], out_hbm.at[pl.ds(s0 + j * CH, CH)],
                    store_sem.at[sl])
        for j in range(njobs - 3, njobs):
            sl = j % 3
            pltpu.make_async_copy(
                xbuf.at[sl], out_hbm.at[pl.ds(s0 + j * CH, CH)],
                store_sem.at[sl]).wait()

    return k(x2, pe_weight)


def kernel(x, pe_weight):
    B, S, D = x.shape
    x2 = x.reshape(B * S, D)
    sc_out = _sc_part(x2, pe_weight, (B - 1) * S, S, D)
    tc_out = _tc_part(x, pe_weight, B - 1)
    return jnp.concatenate([tc_out, sc_out.reshape(1, S, D)], axis=0)


# speedup vs baseline: 2.5618x; 2.5618x over previous
import jax
import jax.numpy as jnp
from jax.experimental import pallas as pl
from jax.experimental.pallas import tpu as pltpu

S_BLK = 512


def _add_pe_kernel(x_ref, pe_ref, o_ref):
    o_ref[...] = x_ref[...] + pe_ref[...][None, :, :]


def kernel(x, pe_weight):
    B, S, D = x.shape
    grid = (S // S_BLK,)
    return pl.pallas_call(
        _add_pe_kernel,
        grid=grid,
        in_specs=[
            pl.BlockSpec((B, S_BLK, D), lambda i: (0, i, 0)),
            pl.BlockSpec((S_BLK, D), lambda i: (i, 0)),
        ],
        out_specs=pl.BlockSpec((B, S_BLK, D), lambda i: (0, i, 0)),
        out_shape=jax.ShapeDtypeStruct((B, S, D), x.dtype),
    )(x, pe_weight)


# speedup vs baseline: 2.5851x; 1.0091x over previous
import jax
import jax.numpy as jnp
from jax.experimental import pallas as pl
from jax.experimental.pallas import tpu as pltpu

S_BLK = 640


def _add_pe_kernel(x_ref, pe_ref, o_ref):
    o_ref[...] = x_ref[...] + pe_ref[...][None, :, :]


def kernel(x, pe_weight):
    B, S, D = x.shape
    grid = (pl.cdiv(S, S_BLK),)
    return pl.pallas_call(
        _add_pe_kernel,
        grid=grid,
        in_specs=[
            pl.BlockSpec((B, S_BLK, D), lambda i: (0, i, 0)),
            pl.BlockSpec((S_BLK, D), lambda i: (i, 0)),
        ],
        out_specs=pl.BlockSpec((B, S_BLK, D), lambda i: (0, i, 0)),
        out_shape=jax.ShapeDtypeStruct((B, S, D), x.dtype),
    )(x, pe_weight)
